# baseline (speedup 1.0000x reference)
import numpy as np
import jax
import jax.numpy as jnp
from jax.experimental import pallas as pl
from jax.experimental.pallas import tpu as pltpu

D = 192
NH = 8
DH = D // NH
WW = 7
WS = WW * WW
TK = 8
NW = 256
ROWS_PER_BLK = 8 * WS
N_BLKS = (NW * WS) // ROWS_PER_BLK

_BIG = np.int32(1 << 30)


def _dotT(a, b):
    return jax.lax.dot_general(
        a.astype(jnp.bfloat16), b.astype(jnp.bfloat16),
        (((1,), (1,)), ((), ())), preferred_element_type=jnp.float32)


def _proj_kernel(x_ref, s_ref, wq_ref, wk_ref, wv_ref,
                 q_ref, k_ref, v_ref, qm_ref, km_ref, vm_ref):
    xb = x_ref[...]
    sb = s_ref[...]
    q = _dotT(xb, wq_ref[...])
    k = _dotT(sb, wk_ref[...])
    v = _dotT(sb, wv_ref[...])
    q_ref[...] = q
    k_ref[...] = k
    v_ref[...] = v
    rows = jax.lax.broadcasted_iota(jnp.int32, (8, ROWS_PER_BLK), 0)
    cols = jax.lax.broadcasted_iota(jnp.int32, (8, ROWS_PER_BLK), 1)
    ind = jnp.where(cols // WS == rows, 1.0, 0.0).astype(jnp.float32)
    inv = np.float32(1.0 / WS)
    qm_ref[...] = inv * jax.lax.dot_general(
        ind, q, (((1,), (0,)), ((), ())), preferred_element_type=jnp.float32,
        precision=jax.lax.Precision.HIGHEST)
    km_ref[...] = inv * jax.lax.dot_general(
        ind, k, (((1,), (0,)), ((), ())), preferred_element_type=jnp.float32,
        precision=jax.lax.Precision.HIGHEST)
    vm_ref[...] = inv * jax.lax.dot_general(
        ind, v, (((1,), (0,)), ((), ())), preferred_element_type=jnp.float32,
        precision=jax.lax.Precision.HIGHEST)


def _topk_kernel(km_ref, qm_ref, idx_ref):
    simT = _dotT(km_ref[...], qm_ref[...])
    row = jax.lax.broadcasted_iota(jnp.int32, (NW, NW), 0)
    for j in range(TK):
        mx = jnp.max(simT, axis=0, keepdims=True)
        cand = jnp.where(simT == mx, row, _BIG)
        am = jnp.min(cand, axis=0, keepdims=True)
        idx_ref[j:j + 1, :] = am
        simT = jnp.where(row == am, -jnp.inf, simT)


def _layer_norm(t, g, b, eps=1e-5):
    mu = jnp.mean(t, axis=-1, keepdims=True)
    var = jnp.mean(jnp.square(t - mu), axis=-1, keepdims=True)
    return (t - mu) / jnp.sqrt(var + eps) * g + b


def _attn_kernel(idx_sref, q_ref, x_ref, kw_ref, vw_ref, km_ref, vm_ref,
                 wm_ref, w1_ref, w2_ref, g1_ref, b1_ref, g2_ref, b2_ref,
                 out_ref):
    i = pl.program_id(0)
    q = q_ref[0]
    ks = [kw_ref[idx_sref[j, i]] for j in range(TK)]
    vs = [vw_ref[idx_sref[j, i]] for j in range(TK)]
    keys = jnp.concatenate(ks + [km_ref[...]], axis=0)
    vals = jnp.concatenate(vs + [vm_ref[...]], axis=0)
    scale = np.float32(1.0 / np.sqrt(DH))
    msg_cols = []
    for h in range(NH):
        sl = slice(h * DH, (h + 1) * DH)
        qk = _dotT(q[:, sl], keys[:, sl]) * scale
        qk = qk - jnp.max(qk, axis=1, keepdims=True)
        e = jnp.exp(qk)
        attn = e / jnp.sum(e, axis=1, keepdims=True)
        msg_cols.append(
            jax.lax.dot_general(attn.astype(jnp.bfloat16),
                                vals[:, sl].astype(jnp.bfloat16),
                                (((1,), (0,)), ((), ())),
                                preferred_element_type=jnp.float32))
    msg = jnp.concatenate(msg_cols, axis=1)
    msg = _dotT(msg, wm_ref[...])
    msg = _layer_norm(msg, g1_ref[...], b1_ref[...])
    hcat = jnp.concatenate([x_ref[0], msg], axis=1)
    h1 = jnp.maximum(_dotT(hcat, w1_ref[...]), 0.0)
    h2 = _dotT(h1, w2_ref[...])
    out_ref[0] = x_ref[0] + _layer_norm(h2, g2_ref[...], b2_ref[...])


def kernel(x, source, Wq, Wk, Wv, Wm, W1, W2, g1, b1, g2, b2):
    b, d, H, Wd = x.shape
    m, n = H // WW, Wd // WW

    def to_win(t):
        t = t.reshape(d, m, WW, n, WW)
        t = jnp.transpose(t, (1, 3, 2, 4, 0))
        return t.reshape(NW * WS, d)

    xw = to_win(x[0])
    sw = to_win(source[0])

    blk = lambda i: (i, 0)
    qf, kf, vf, qm, km, vm = pl.pallas_call(
        _proj_kernel,
        grid=(N_BLKS,),
        in_specs=[
            pl.BlockSpec((ROWS_PER_BLK, D), blk),
            pl.BlockSpec((ROWS_PER_BLK, D), blk),
            pl.BlockSpec((D, D), lambda i: (0, 0)),
            pl.BlockSpec((D, D), lambda i: (0, 0)),
            pl.BlockSpec((D, D), lambda i: (0, 0)),
        ],
        out_specs=[
            pl.BlockSpec((ROWS_PER_BLK, D), blk),
            pl.BlockSpec((ROWS_PER_BLK, D), blk),
            pl.BlockSpec((ROWS_PER_BLK, D), blk),
            pl.BlockSpec((8, D), blk),
            pl.BlockSpec((8, D), blk),
            pl.BlockSpec((8, D), blk),
        ],
        out_shape=[
            jax.ShapeDtypeStruct((NW * WS, D), jnp.float32),
            jax.ShapeDtypeStruct((NW * WS, D), jnp.float32),
            jax.ShapeDtypeStruct((NW * WS, D), jnp.float32),
            jax.ShapeDtypeStruct((NW, D), jnp.float32),
            jax.ShapeDtypeStruct((NW, D), jnp.float32),
            jax.ShapeDtypeStruct((NW, D), jnp.float32),
        ],
    )(xw, sw, Wq, Wk, Wv)

    idx = pl.pallas_call(
        _topk_kernel,
        out_shape=jax.ShapeDtypeStruct((TK, NW), jnp.int32),
    )(km, qm)

    full3 = lambda i, s: (0, 0, 0)
    full2 = lambda i, s: (0, 0)
    out_w = pl.pallas_call(
        _attn_kernel,
        grid_spec=pltpu.PrefetchScalarGridSpec(
            num_scalar_prefetch=1,
            grid=(NW,),
            in_specs=[
                pl.BlockSpec((1, WS, D), lambda i, s: (i, 0, 0)),
                pl.BlockSpec((1, WS, D), lambda i, s: (i, 0, 0)),
                pl.BlockSpec((NW, WS, D), full3),
                pl.BlockSpec((NW, WS, D), full3),
                pl.BlockSpec((NW, D), full2),
                pl.BlockSpec((NW, D), full2),
                pl.BlockSpec((D, D), full2),
                pl.BlockSpec((2 * D, 2 * D), full2),
                pl.BlockSpec((D, 2 * D), full2),
                pl.BlockSpec((1, D), full2),
                pl.BlockSpec((1, D), full2),
                pl.BlockSpec((1, D), full2),
                pl.BlockSpec((1, D), full2),
            ],
            out_specs=pl.BlockSpec((1, WS, D), lambda i, s: (i, 0, 0)),
        ),
        out_shape=jax.ShapeDtypeStruct((NW, WS, D), jnp.float32),
    )(idx,
      qf.reshape(NW, WS, D), xw.reshape(NW, WS, D),
      kf.reshape(NW, WS, D), vf.reshape(NW, WS, D),
      km, vm, Wm, W1, W2,
      g1.reshape(1, D), b1.reshape(1, D), g2.reshape(1, D), b2.reshape(1, D))

    out = out_w.reshape(m, n, WW, WW, d)
    out = jnp.transpose(out, (4, 0, 2, 1, 3)).reshape(1, d, H, Wd)
    return out


# speedup vs baseline: 1.7845x; 1.7845x over previous
import numpy as np
import jax
import jax.numpy as jnp
from jax.experimental import pallas as pl
from jax.experimental.pallas import tpu as pltpu

D = 192
NH = 8
DH = D // NH
WW = 7
WS = WW * WW
TK = 8
NW = 256
NKEY = TK * WS + NW
ROWS_PER_BLK = 8 * WS
N_BLKS = (NW * WS) // ROWS_PER_BLK

_BIG = np.int32(1 << 30)


def _dotTbf(a, b):
    return jax.lax.dot_general(
        a.astype(jnp.bfloat16), b.astype(jnp.bfloat16),
        (((1,), (1,)), ((), ())), preferred_element_type=jnp.float32)


def _proj_kernel(x_ref, s_ref, wq_ref, wk_ref, wv_ref,
                 q_ref, k_ref, v_ref, qm_ref, km_ref, vm_ref,
                 kmb_ref, vmb_ref):
    xb = x_ref[...].astype(jnp.bfloat16)
    sb = s_ref[...].astype(jnp.bfloat16)
    q = jax.lax.dot_general(xb, wq_ref[...], (((1,), (1,)), ((), ())),
                            preferred_element_type=jnp.float32)
    k = jax.lax.dot_general(sb, wk_ref[...], (((1,), (1,)), ((), ())),
                            preferred_element_type=jnp.float32)
    v = jax.lax.dot_general(sb, wv_ref[...], (((1,), (1,)), ((), ())),
                            preferred_element_type=jnp.float32)
    q_ref[...] = q.astype(jnp.bfloat16)
    k_ref[...] = k.astype(jnp.bfloat16)
    v_ref[...] = v.astype(jnp.bfloat16)
    rows = jax.lax.broadcasted_iota(jnp.int32, (8, ROWS_PER_BLK), 0)
    cols = jax.lax.broadcasted_iota(jnp.int32, (8, ROWS_PER_BLK), 1)
    ind = jnp.where(cols // WS == rows, 1.0, 0.0).astype(jnp.float32)
    inv = np.float32(1.0 / WS)
    qm = inv * jax.lax.dot_general(
        ind, q, (((1,), (0,)), ((), ())), preferred_element_type=jnp.float32,
        precision=jax.lax.Precision.HIGHEST)
    km = inv * jax.lax.dot_general(
        ind, k, (((1,), (0,)), ((), ())), preferred_element_type=jnp.float32,
        precision=jax.lax.Precision.HIGHEST)
    vm = inv * jax.lax.dot_general(
        ind, v, (((1,), (0,)), ((), ())), preferred_element_type=jnp.float32,
        precision=jax.lax.Precision.HIGHEST)
    qm_ref[...] = qm
    km_ref[...] = km
    vm_ref[...] = vm
    kmb_ref[...] = km.astype(jnp.bfloat16)
    vmb_ref[...] = vm.astype(jnp.bfloat16)


def _topk_kernel(km_ref, qm_ref, idx_ref):
    simT = _dotTbf(km_ref[...], qm_ref[...])
    row = jax.lax.broadcasted_iota(jnp.int32, (NW, NW), 0)
    for j in range(TK):
        mx = jnp.max(simT, axis=0, keepdims=True)
        cand = jnp.where(simT == mx, row, _BIG)
        am = jnp.min(cand, axis=0, keepdims=True)
        idx_ref[j:j + 1, :] = am
        simT = jnp.where(row == am, -jnp.inf, simT)


def _layer_norm(t, g, b, eps=1e-5):
    mu = jnp.mean(t, axis=-1, keepdims=True)
    var = jnp.mean(jnp.square(t - mu), axis=-1, keepdims=True)
    return (t - mu) / jnp.sqrt(var + eps) * g + b


def _attn_kernel(idx_sref, q_ref, x_ref, kw_ref, vw_ref, km_ref, vm_ref,
                 wm_ref, w1_ref, w2_ref, g1_ref, b1_ref, g2_ref, b2_ref,
                 out_ref):
    i = pl.program_id(0)
    q = q_ref[0]
    ks = [kw_ref[idx_sref[j, i]] for j in range(TK)]
    vs = [vw_ref[idx_sref[j, i]] for j in range(TK)]
    keys = jnp.concatenate(ks + [km_ref[...]], axis=0)
    vals = jnp.concatenate(vs + [vm_ref[...]], axis=0)
    lane = jax.lax.broadcasted_iota(jnp.int32, (NH * WS, D), 1) // DH
    rowh = jax.lax.broadcasted_iota(jnp.int32, (NH * WS, D), 0) // WS
    qtile = jnp.concatenate([q] * NH, axis=0)
    q_stack = jnp.where(lane == rowh, qtile, jnp.bfloat16(0.0))
    scale = np.float32(1.0 / np.sqrt(DH))
    qk = jax.lax.dot_general(q_stack, keys, (((1,), (1,)), ((), ())),
                             preferred_element_type=jnp.float32) * scale
    qk = qk - jnp.max(qk, axis=1, keepdims=True)
    e = jnp.exp(qk)
    attn = (e / jnp.sum(e, axis=1, keepdims=True)).astype(jnp.bfloat16)
    pv = jax.lax.dot_general(attn, vals, (((1,), (0,)), ((), ())),
                             preferred_element_type=jnp.float32)
    lane49 = jax.lax.broadcasted_iota(jnp.int32, (WS, D), 1) // DH
    msg = jnp.zeros((WS, D), jnp.float32)
    for h in range(NH):
        msg = msg + jnp.where(lane49 == h, pv[h * WS:(h + 1) * WS, :], 0.0)
    xb = x_ref[0]
    msg = _dotTbf(msg, wm_ref[...])
    msg = _layer_norm(msg, g1_ref[...], b1_ref[...])
    hcat = jnp.concatenate([xb, msg], axis=1)
    h1 = jnp.maximum(_dotTbf(hcat, w1_ref[...]), 0.0)
    h2 = _dotTbf(h1, w2_ref[...])
    out_ref[0] = xb + _layer_norm(h2, g2_ref[...], b2_ref[...])


def kernel(x, source, Wq, Wk, Wv, Wm, W1, W2, g1, b1, g2, b2):
    b, d, H, Wd = x.shape
    m, n = H // WW, Wd // WW

    def to_win(t):
        t = t.reshape(d, m, WW, n, WW)
        t = jnp.transpose(t, (1, 3, 2, 4, 0))
        return t.reshape(NW * WS, d)

    xw = to_win(x[0])
    sw = to_win(source[0])
    bf = jnp.bfloat16

    blk = lambda i: (i, 0)
    qf, kf, vf, qm, km, vm, kmb, vmb = pl.pallas_call(
        _proj_kernel,
        grid=(N_BLKS,),
        in_specs=[
            pl.BlockSpec((ROWS_PER_BLK, D), blk),
            pl.BlockSpec((ROWS_PER_BLK, D), blk),
            pl.BlockSpec((D, D), lambda i: (0, 0)),
            pl.BlockSpec((D, D), lambda i: (0, 0)),
            pl.BlockSpec((D, D), lambda i: (0, 0)),
        ],
        out_specs=[
            pl.BlockSpec((ROWS_PER_BLK, D), blk),
            pl.BlockSpec((ROWS_PER_BLK, D), blk),
            pl.BlockSpec((ROWS_PER_BLK, D), blk),
            pl.BlockSpec((8, D), blk),
            pl.BlockSpec((8, D), blk),
            pl.BlockSpec((8, D), blk),
            pl.BlockSpec((8, D), blk),
            pl.BlockSpec((8, D), blk),
        ],
        out_shape=[
            jax.ShapeDtypeStruct((NW * WS, D), bf),
            jax.ShapeDtypeStruct((NW * WS, D), bf),
            jax.ShapeDtypeStruct((NW * WS, D), bf),
            jax.ShapeDtypeStruct((NW, D), jnp.float32),
            jax.ShapeDtypeStruct((NW, D), jnp.float32),
            jax.ShapeDtypeStruct((NW, D), jnp.float32),
            jax.ShapeDtypeStruct((NW, D), bf),
            jax.ShapeDtypeStruct((NW, D), bf),
        ],
    )(xw, sw, Wq.astype(bf), Wk.astype(bf), Wv.astype(bf))

    idx = pl.pallas_call(
        _topk_kernel,
        out_shape=jax.ShapeDtypeStruct((TK, NW), jnp.int32),
    )(km, qm)

    full3 = lambda i, s: (0, 0, 0)
    full2 = lambda i, s: (0, 0)
    out_w = pl.pallas_call(
        _attn_kernel,
        grid_spec=pltpu.PrefetchScalarGridSpec(
            num_scalar_prefetch=1,
            grid=(NW,),
            in_specs=[
                pl.BlockSpec((1, WS, D), lambda i, s: (i, 0, 0)),
                pl.BlockSpec((1, WS, D), lambda i, s: (i, 0, 0)),
                pl.BlockSpec((NW, WS, D), full3),
                pl.BlockSpec((NW, WS, D), full3),
                pl.BlockSpec((NW, D), full2),
                pl.BlockSpec((NW, D), full2),
                pl.BlockSpec((D, D), full2),
                pl.BlockSpec((2 * D, 2 * D), full2),
                pl.BlockSpec((D, 2 * D), full2),
                pl.BlockSpec((1, D), full2),
                pl.BlockSpec((1, D), full2),
                pl.BlockSpec((1, D), full2),
                pl.BlockSpec((1, D), full2),
            ],
            out_specs=pl.BlockSpec((1, WS, D), lambda i, s: (i, 0, 0)),
        ),
        out_shape=jax.ShapeDtypeStruct((NW, WS, D), jnp.float32),
    )(idx,
      qf.reshape(NW, WS, D), xw.reshape(NW, WS, D),
      kf.reshape(NW, WS, D), vf.reshape(NW, WS, D),
      kmb, vmb, Wm.astype(bf), W1.astype(bf), W2.astype(bf),
      g1.reshape(1, D), b1.reshape(1, D), g2.reshape(1, D), b2.reshape(1, D))

    out = out_w.reshape(m, n, WW, WW, d)
    out = jnp.transpose(out, (4, 0, 2, 1, 3)).reshape(1, d, H, Wd)
    return out


# speedup vs baseline: 2.1774x; 1.2202x over previous
import numpy as np
import jax
import jax.numpy as jnp
from jax.experimental import pallas as pl
from jax.experimental.pallas import tpu as pltpu

D = 192
NH = 8
DH = D // NH
WW = 7
WS = WW * WW
TK = 8
NW = 256
NKEY = TK * WS + NW
ROWS_PER_BLK = 8 * WS
N_BLKS = (NW * WS) // ROWS_PER_BLK

_BIG = np.int32(1 << 30)


def _dotTbf(a, b):
    return jax.lax.dot_general(
        a.astype(jnp.bfloat16), b.astype(jnp.bfloat16),
        (((1,), (1,)), ((), ())), preferred_element_type=jnp.float32)


def _proj_kernel(x_ref, s_ref, wq_ref, wk_ref, wv_ref,
                 q_ref, k_ref, v_ref, qm_ref, km_ref, vm_ref,
                 kmb_ref, vmb_ref):
    xb = x_ref[...].astype(jnp.bfloat16)
    sb = s_ref[...].astype(jnp.bfloat16)
    q = jax.lax.dot_general(xb, wq_ref[...], (((1,), (1,)), ((), ())),
                            preferred_element_type=jnp.float32)
    k = jax.lax.dot_general(sb, wk_ref[...], (((1,), (1,)), ((), ())),
                            preferred_element_type=jnp.float32)
    v = jax.lax.dot_general(sb, wv_ref[...], (((1,), (1,)), ((), ())),
                            preferred_element_type=jnp.float32)
    q_ref[...] = q.astype(jnp.bfloat16)
    k_ref[...] = k.astype(jnp.bfloat16)
    v_ref[...] = v.astype(jnp.bfloat16)
    rows = jax.lax.broadcasted_iota(jnp.int32, (8, ROWS_PER_BLK), 0)
    cols = jax.lax.broadcasted_iota(jnp.int32, (8, ROWS_PER_BLK), 1)
    ind = jnp.where(cols // WS == rows, 1.0, 0.0).astype(jnp.float32)
    inv = np.float32(1.0 / WS)
    qm = inv * jax.lax.dot_general(
        ind, q, (((1,), (0,)), ((), ())), preferred_element_type=jnp.float32,
        precision=jax.lax.Precision.HIGHEST)
    km = inv * jax.lax.dot_general(
        ind, k, (((1,), (0,)), ((), ())), preferred_element_type=jnp.float32,
        precision=jax.lax.Precision.HIGHEST)
    vm = inv * jax.lax.dot_general(
        ind, v, (((1,), (0,)), ((), ())), preferred_element_type=jnp.float32,
        precision=jax.lax.Precision.HIGHEST)
    qm_ref[...] = qm
    km_ref[...] = km
    vm_ref[...] = vm
    kmb_ref[...] = km.astype(jnp.bfloat16)
    vmb_ref[...] = vm.astype(jnp.bfloat16)


def _topk_kernel(km_ref, qm_ref, idx_ref):
    simT = _dotTbf(km_ref[...], qm_ref[...])
    row = jax.lax.broadcasted_iota(jnp.int32, (NW, NW), 0)
    for j in range(TK):
        mx = jnp.max(simT, axis=0, keepdims=True)
        cand = jnp.where(simT == mx, row, _BIG)
        am = jnp.min(cand, axis=0, keepdims=True)
        idx_ref[j:j + 1, :] = am
        simT = jnp.where(row == am, -jnp.inf, simT)


def _layer_norm(t, g, b, eps=1e-5):
    mu = jnp.mean(t, axis=-1, keepdims=True)
    var = jnp.mean(jnp.square(t - mu), axis=-1, keepdims=True)
    return (t - mu) / jnp.sqrt(var + eps) * g + b


WPS = 2


def _attn_kernel(idx_sref, q_ref, x_ref, kw_ref, vw_ref, km_ref, vm_ref,
                 wm_ref, w1_ref, w2_ref, g1_ref, b1_ref, g2_ref, b2_ref,
                 out_ref):
    i = pl.program_id(0)
    lane = jax.lax.broadcasted_iota(jnp.int32, (NH * WS, D), 1) // DH
    rowh = jax.lax.broadcasted_iota(jnp.int32, (NH * WS, D), 0) // WS
    lane49 = jax.lax.broadcasted_iota(jnp.int32, (WS, D), 1) // DH
    scale = np.float32(1.0 / np.sqrt(DH))
    for w in range(WPS):
        win = i * WPS + w
        q = q_ref[w]
        ks = [kw_ref[idx_sref[j, win]] for j in range(TK)]
        vs = [vw_ref[idx_sref[j, win]] for j in range(TK)]
        keys = jnp.concatenate(ks + [km_ref[...]], axis=0)
        vals = jnp.concatenate(vs + [vm_ref[...]], axis=0)
        qtile = jnp.concatenate([q] * NH, axis=0)
        q_stack = jnp.where(lane == rowh, qtile, jnp.bfloat16(0.0))
        qk = jax.lax.dot_general(q_stack, keys, (((1,), (1,)), ((), ())),
                                 preferred_element_type=jnp.float32)
        e = jnp.exp(qk * scale)
        s = jnp.sum(e, axis=1, keepdims=True)
        pv = jax.lax.dot_general(e.astype(jnp.bfloat16), vals,
                                 (((1,), (0,)), ((), ())),
                                 preferred_element_type=jnp.float32)
        pv = pv * (1.0 / s)
        msg = jnp.zeros((WS, D), jnp.float32)
        for h in range(NH):
            msg = msg + jnp.where(lane49 == h,
                                  pv[h * WS:(h + 1) * WS, :], 0.0)
        xb = x_ref[w]
        msg = _dotTbf(msg, wm_ref[...])
        msg = _layer_norm(msg, g1_ref[...], b1_ref[...])
        hcat = jnp.concatenate([xb, msg], axis=1)
        h1 = jnp.maximum(_dotTbf(hcat, w1_ref[...]), 0.0)
        h2 = _dotTbf(h1, w2_ref[...])
        out_ref[w] = xb + _layer_norm(h2, g2_ref[...], b2_ref[...])


def kernel(x, source, Wq, Wk, Wv, Wm, W1, W2, g1, b1, g2, b2):
    b, d, H, Wd = x.shape
    m, n = H // WW, Wd // WW

    def to_win(t):
        t = t.reshape(d, m, WW, n, WW)
        t = jnp.transpose(t, (1, 3, 2, 4, 0))
        return t.reshape(NW * WS, d)

    xw = to_win(x[0])
    sw = to_win(source[0])
    bf = jnp.bfloat16

    blk = lambda i: (i, 0)
    qf, kf, vf, qm, km, vm, kmb, vmb = pl.pallas_call(
        _proj_kernel,
        grid=(N_BLKS,),
        in_specs=[
            pl.BlockSpec((ROWS_PER_BLK, D), blk),
            pl.BlockSpec((ROWS_PER_BLK, D), blk),
            pl.BlockSpec((D, D), lambda i: (0, 0)),
            pl.BlockSpec((D, D), lambda i: (0, 0)),
            pl.BlockSpec((D, D), lambda i: (0, 0)),
        ],
        out_specs=[
            pl.BlockSpec((ROWS_PER_BLK, D), blk),
            pl.BlockSpec((ROWS_PER_BLK, D), blk),
            pl.BlockSpec((ROWS_PER_BLK, D), blk),
            pl.BlockSpec((8, D), blk),
            pl.BlockSpec((8, D), blk),
            pl.BlockSpec((8, D), blk),
            pl.BlockSpec((8, D), blk),
            pl.BlockSpec((8, D), blk),
        ],
        out_shape=[
            jax.ShapeDtypeStruct((NW * WS, D), bf),
            jax.ShapeDtypeStruct((NW * WS, D), bf),
            jax.ShapeDtypeStruct((NW * WS, D), bf),
            jax.ShapeDtypeStruct((NW, D), jnp.float32),
            jax.ShapeDtypeStruct((NW, D), jnp.float32),
            jax.ShapeDtypeStruct((NW, D), jnp.float32),
            jax.ShapeDtypeStruct((NW, D), bf),
            jax.ShapeDtypeStruct((NW, D), bf),
        ],
    )(xw, sw, Wq.astype(bf), Wk.astype(bf), Wv.astype(bf))

    idx = pl.pallas_call(
        _topk_kernel,
        out_shape=jax.ShapeDtypeStruct((TK, NW), jnp.int32),
    )(km, qm)

    full3 = lambda i, s: (0, 0, 0)
    full2 = lambda i, s: (0, 0)
    out_w = pl.pallas_call(
        _attn_kernel,
        grid_spec=pltpu.PrefetchScalarGridSpec(
            num_scalar_prefetch=1,
            grid=(NW // WPS,),
            in_specs=[
                pl.BlockSpec((WPS, WS, D), lambda i, s: (i, 0, 0)),
                pl.BlockSpec((WPS, WS, D), lambda i, s: (i, 0, 0)),
                pl.BlockSpec((NW, WS, D), full3),
                pl.BlockSpec((NW, WS, D), full3),
                pl.BlockSpec((NW, D), full2),
                pl.BlockSpec((NW, D), full2),
                pl.BlockSpec((D, D), full2),
                pl.BlockSpec((2 * D, 2 * D), full2),
                pl.BlockSpec((D, 2 * D), full2),
                pl.BlockSpec((1, D), full2),
                pl.BlockSpec((1, D), full2),
                pl.BlockSpec((1, D), full2),
                pl.BlockSpec((1, D), full2),
            ],
            out_specs=pl.BlockSpec((WPS, WS, D), lambda i, s: (i, 0, 0)),
        ),
        out_shape=jax.ShapeDtypeStruct((NW, WS, D), jnp.float32),
    )(idx,
      qf.reshape(NW, WS, D), xw.reshape(NW, WS, D),
      kf.reshape(NW, WS, D), vf.reshape(NW, WS, D),
      kmb, vmb, Wm.astype(bf), W1.astype(bf), W2.astype(bf),
      g1.reshape(1, D), b1.reshape(1, D), g2.reshape(1, D), b2.reshape(1, D))

    out = out_w.reshape(m, n, WW, WW, d)
    out = jnp.transpose(out, (4, 0, 2, 1, 3)).reshape(1, d, H, Wd)
    return out


# speedup vs baseline: 3.2598x; 1.4971x over previous
import numpy as np
import jax
import jax.numpy as jnp
from jax.experimental import pallas as pl
from jax.experimental.pallas import tpu as pltpu

D = 192
NH = 8
DH = D // NH
WW = 7
WS = WW * WW
TK = 8
NW = 256
NKEY = TK * WS + NW
ROWS_PER_BLK = 8 * WS
N_BLKS = (NW * WS) // ROWS_PER_BLK

_BIG = np.int32(1 << 30)


def _dotTbf(a, b):
    return jax.lax.dot_general(
        a.astype(jnp.bfloat16), b.astype(jnp.bfloat16),
        (((1,), (1,)), ((), ())), preferred_element_type=jnp.float32)


def _proj_kernel(x_ref, s_ref, wq_ref, wk_ref, wv_ref,
                 q_ref, k_ref, v_ref, qm_ref, km_ref, vm_ref,
                 kmb_ref, vmb_ref):
    xb = x_ref[...].astype(jnp.bfloat16)
    sb = s_ref[...].astype(jnp.bfloat16)
    q = jax.lax.dot_general(xb, wq_ref[...], (((1,), (1,)), ((), ())),
                            preferred_element_type=jnp.float32)
    k = jax.lax.dot_general(sb, wk_ref[...], (((1,), (1,)), ((), ())),
                            preferred_element_type=jnp.float32)
    v = jax.lax.dot_general(sb, wv_ref[...], (((1,), (1,)), ((), ())),
                            preferred_element_type=jnp.float32)
    q_ref[...] = q.astype(jnp.bfloat16)
    k_ref[...] = k.astype(jnp.bfloat16)
    v_ref[...] = v.astype(jnp.bfloat16)
    rows = jax.lax.broadcasted_iota(jnp.int32, (8, ROWS_PER_BLK), 0)
    cols = jax.lax.broadcasted_iota(jnp.int32, (8, ROWS_PER_BLK), 1)
    ind = jnp.where(cols // WS == rows, 1.0, 0.0).astype(jnp.float32)
    inv = np.float32(1.0 / WS)
    qm = inv * jax.lax.dot_general(
        ind, q, (((1,), (0,)), ((), ())), preferred_element_type=jnp.float32,
        precision=jax.lax.Precision.HIGHEST)
    km = inv * jax.lax.dot_general(
        ind, k, (((1,), (0,)), ((), ())), preferred_element_type=jnp.float32,
        precision=jax.lax.Precision.HIGHEST)
    vm = inv * jax.lax.dot_general(
        ind, v, (((1,), (0,)), ((), ())), preferred_element_type=jnp.float32,
        precision=jax.lax.Precision.HIGHEST)
    qm_ref[...] = qm
    km_ref[...] = km
    vm_ref[...] = vm
    kmb_ref[...] = km.astype(jnp.bfloat16)
    vmb_ref[...] = vm.astype(jnp.bfloat16)


def _topk_kernel(km_ref, qm_ref, idx_ref):
    simT = _dotTbf(km_ref[...], qm_ref[...])
    row = jax.lax.broadcasted_iota(jnp.int32, (NW, NW), 0)
    for j in range(TK):
        mx = jnp.max(simT, axis=0, keepdims=True)
        cand = jnp.where(simT == mx, row, _BIG)
        am = jnp.min(cand, axis=0, keepdims=True)
        idx_ref[j:j + 1, :] = am
        simT = jnp.where(row == am, -jnp.inf, simT)


def _layer_norm(t, g, b, eps=1e-5):
    mu = jnp.mean(t, axis=-1, keepdims=True)
    var = jnp.mean(jnp.square(t - mu), axis=-1, keepdims=True)
    return (t - mu) / jnp.sqrt(var + eps) * g + b


WPS = 8


def _attn_kernel(idx_sref, q_ref, x_ref, kw_ref, vw_ref, km_ref, vm_ref,
                 wm_ref, w1_ref, w2_ref, g1_ref, b1_ref, g2_ref, b2_ref,
                 out_ref):
    i = pl.program_id(0)
    lane = jax.lax.broadcasted_iota(jnp.int32, (NH * WS, D), 1) // DH
    rowh = jax.lax.broadcasted_iota(jnp.int32, (NH * WS, D), 0) // WS
    lane49 = jax.lax.broadcasted_iota(jnp.int32, (WS, D), 1) // DH
    scale = np.float32(1.0 / np.sqrt(DH))
    msgs = []
    for w in range(WPS):
        win = i * WPS + w
        q = q_ref[w * WS:(w + 1) * WS, :]
        ks = [kw_ref[idx_sref[j, win]] for j in range(TK)]
        vs = [vw_ref[idx_sref[j, win]] for j in range(TK)]
        keys = jnp.concatenate(ks + [km_ref[...]], axis=0)
        vals = jnp.concatenate(vs + [vm_ref[...]], axis=0)
        qtile = jnp.concatenate([q] * NH, axis=0)
        q_stack = jnp.where(lane == rowh, qtile, jnp.bfloat16(0.0))
        qk = jax.lax.dot_general(q_stack, keys, (((1,), (1,)), ((), ())),
                                 preferred_element_type=jnp.float32)
        e = jnp.exp(qk * scale)
        s = jnp.sum(e, axis=1, keepdims=True)
        pv = jax.lax.dot_general(e.astype(jnp.bfloat16), vals,
                                 (((1,), (0,)), ((), ())),
                                 preferred_element_type=jnp.float32)
        pv = pv * (1.0 / s)
        msg = jnp.zeros((WS, D), jnp.float32)
        for h in range(NH):
            msg = msg + jnp.where(lane49 == h,
                                  pv[h * WS:(h + 1) * WS, :], 0.0)
        msgs.append(msg)
    msg = jnp.concatenate(msgs, axis=0)
    xb = x_ref[...]
    msg = _dotTbf(msg, wm_ref[...])
    msg = _layer_norm(msg, g1_ref[...], b1_ref[...])
    hcat = jnp.concatenate([xb, msg], axis=1)
    h1 = jnp.maximum(_dotTbf(hcat, w1_ref[...]), 0.0)
    h2 = _dotTbf(h1, w2_ref[...])
    out_ref[...] = xb + _layer_norm(h2, g2_ref[...], b2_ref[...])


def kernel(x, source, Wq, Wk, Wv, Wm, W1, W2, g1, b1, g2, b2):
    b, d, H, Wd = x.shape
    m, n = H // WW, Wd // WW

    def to_win(t):
        t = t.reshape(d, m, WW, n, WW)
        t = jnp.transpose(t, (1, 3, 2, 4, 0))
        return t.reshape(NW * WS, d)

    xw = to_win(x[0])
    sw = to_win(source[0])
    bf = jnp.bfloat16

    blk = lambda i: (i, 0)
    qf, kf, vf, qm, km, vm, kmb, vmb = pl.pallas_call(
        _proj_kernel,
        grid=(N_BLKS,),
        in_specs=[
            pl.BlockSpec((ROWS_PER_BLK, D), blk),
            pl.BlockSpec((ROWS_PER_BLK, D), blk),
            pl.BlockSpec((D, D), lambda i: (0, 0)),
            pl.BlockSpec((D, D), lambda i: (0, 0)),
            pl.BlockSpec((D, D), lambda i: (0, 0)),
        ],
        out_specs=[
            pl.BlockSpec((ROWS_PER_BLK, D), blk),
            pl.BlockSpec((ROWS_PER_BLK, D), blk),
            pl.BlockSpec((ROWS_PER_BLK, D), blk),
            pl.BlockSpec((8, D), blk),
            pl.BlockSpec((8, D), blk),
            pl.BlockSpec((8, D), blk),
            pl.BlockSpec((8, D), blk),
            pl.BlockSpec((8, D), blk),
        ],
        out_shape=[
            jax.ShapeDtypeStruct((NW * WS, D), bf),
            jax.ShapeDtypeStruct((NW * WS, D), bf),
            jax.ShapeDtypeStruct((NW * WS, D), bf),
            jax.ShapeDtypeStruct((NW, D), jnp.float32),
            jax.ShapeDtypeStruct((NW, D), jnp.float32),
            jax.ShapeDtypeStruct((NW, D), jnp.float32),
            jax.ShapeDtypeStruct((NW, D), bf),
            jax.ShapeDtypeStruct((NW, D), bf),
        ],
    )(xw, sw, Wq.astype(bf), Wk.astype(bf), Wv.astype(bf))

    idx = pl.pallas_call(
        _topk_kernel,
        out_shape=jax.ShapeDtypeStruct((TK, NW), jnp.int32),
    )(km, qm)

    full2 = lambda i, s: (0, 0)
    rows = WPS * WS
    out_w = pl.pallas_call(
        _attn_kernel,
        grid_spec=pltpu.PrefetchScalarGridSpec(
            num_scalar_prefetch=1,
            grid=(NW // WPS,),
            in_specs=[
                pl.BlockSpec((rows, D), lambda i, s: (i, 0)),
                pl.BlockSpec((rows, D), lambda i, s: (i, 0)),
                pl.BlockSpec((NW, WS, D), lambda i, s: (0, 0, 0)),
                pl.BlockSpec((NW, WS, D), lambda i, s: (0, 0, 0)),
                pl.BlockSpec((NW, D), full2),
                pl.BlockSpec((NW, D), full2),
                pl.BlockSpec((D, D), full2),
                pl.BlockSpec((2 * D, 2 * D), full2),
                pl.BlockSpec((D, 2 * D), full2),
                pl.BlockSpec((1, D), full2),
                pl.BlockSpec((1, D), full2),
                pl.BlockSpec((1, D), full2),
                pl.BlockSpec((1, D), full2),
            ],
            out_specs=pl.BlockSpec((rows, D), lambda i, s: (i, 0)),
        ),
        out_shape=jax.ShapeDtypeStruct((NW * WS, D), jnp.float32),
    )(idx,
      qf, xw, kf.reshape(NW, WS, D), vf.reshape(NW, WS, D),
      kmb, vmb, Wm.astype(bf), W1.astype(bf), W2.astype(bf),
      g1.reshape(1, D), b1.reshape(1, D), g2.reshape(1, D), b2.reshape(1, D))

    out = out_w.reshape(m, n, WW, WW, d)
    out = jnp.transpose(out, (4, 0, 2, 1, 3)).reshape(1, d, H, Wd)
    return out


# speedup vs baseline: 3.3843x; 1.0382x over previous
import numpy as np
import jax
import jax.numpy as jnp
from jax.experimental import pallas as pl
from jax.experimental.pallas import tpu as pltpu

D = 192
NH = 8
DH = D // NH
WW = 7
WS = WW * WW
TK = 8
NW = 256
NKEY = TK * WS + NW
ROWS_PER_BLK = 8 * WS
N_BLKS = (NW * WS) // ROWS_PER_BLK

_BIG = np.int32(1 << 30)


def _dotTbf(a, b):
    return jax.lax.dot_general(
        a.astype(jnp.bfloat16), b.astype(jnp.bfloat16),
        (((1,), (1,)), ((), ())), preferred_element_type=jnp.float32)


def _proj_kernel(x_ref, s_ref, wq_ref, wk_ref, wv_ref,
                 q_ref, k_ref, v_ref, qm_ref, km_ref, vm_ref,
                 kmb_ref, vmb_ref):
    xb = x_ref[...].astype(jnp.bfloat16)
    sb = s_ref[...].astype(jnp.bfloat16)
    q = jax.lax.dot_general(xb, wq_ref[...], (((1,), (1,)), ((), ())),
                            preferred_element_type=jnp.float32)
    k = jax.lax.dot_general(sb, wk_ref[...], (((1,), (1,)), ((), ())),
                            preferred_element_type=jnp.float32)
    v = jax.lax.dot_general(sb, wv_ref[...], (((1,), (1,)), ((), ())),
                            preferred_element_type=jnp.float32)
    q_ref[...] = q.astype(jnp.bfloat16)
    k_ref[...] = k.astype(jnp.bfloat16)
    v_ref[...] = v.astype(jnp.bfloat16)
    rows = jax.lax.broadcasted_iota(jnp.int32, (8, ROWS_PER_BLK), 0)
    cols = jax.lax.broadcasted_iota(jnp.int32, (8, ROWS_PER_BLK), 1)
    ind = jnp.where(cols // WS == rows, 1.0, 0.0).astype(jnp.float32)
    inv = np.float32(1.0 / WS)
    qm = inv * jax.lax.dot_general(
        ind, q, (((1,), (0,)), ((), ())), preferred_element_type=jnp.float32,
        precision=jax.lax.Precision.HIGHEST)
    km = inv * jax.lax.dot_general(
        ind, k, (((1,), (0,)), ((), ())), preferred_element_type=jnp.float32,
        precision=jax.lax.Precision.HIGHEST)
    vm = inv * jax.lax.dot_general(
        ind, v, (((1,), (0,)), ((), ())), preferred_element_type=jnp.float32,
        precision=jax.lax.Precision.HIGHEST)
    qm_ref[...] = qm
    km_ref[...] = km
    vm_ref[...] = vm
    kmb_ref[...] = km.astype(jnp.bfloat16)
    vmb_ref[...] = vm.astype(jnp.bfloat16)


def _topk_kernel(km_ref, qm_ref, idx_ref):
    simT = _dotTbf(km_ref[...], qm_ref[...])
    row = jax.lax.broadcasted_iota(jnp.int32, (NW, NW), 0)
    for j in range(TK):
        mx = jnp.max(simT, axis=0, keepdims=True)
        cand = jnp.where(simT == mx, row, _BIG)
        am = jnp.min(cand, axis=0, keepdims=True)
        idx_ref[j:j + 1, :] = am
        simT = jnp.where(row == am, -jnp.inf, simT)


def _layer_norm(t, g, b, eps=1e-5):
    mu = jnp.mean(t, axis=-1, keepdims=True)
    var = jnp.mean(jnp.square(t - mu), axis=-1, keepdims=True)
    return (t - mu) / jnp.sqrt(var + eps) * g + b


WPS = 16


def _attn_kernel(idx_sref, q_ref, x_ref, kw_ref, vw_ref, km_ref, vm_ref,
                 wm_ref, w1_ref, w2_ref, g1_ref, b1_ref, g2_ref, b2_ref,
                 out_ref):
    i = pl.program_id(0)
    lane = jax.lax.broadcasted_iota(jnp.int32, (NH * WS, D), 1) // DH
    rowh = jax.lax.broadcasted_iota(jnp.int32, (NH * WS, D), 0) // WS
    lane49 = jax.lax.broadcasted_iota(jnp.int32, (WS, D), 1) // DH
    scale = np.float32(1.0 / np.sqrt(DH))
    msgs = []
    for w in range(WPS):
        win = i * WPS + w
        q = q_ref[w * WS:(w + 1) * WS, :]
        ks = [kw_ref[idx_sref[j, win]] for j in range(TK)]
        vs = [vw_ref[idx_sref[j, win]] for j in range(TK)]
        keys = jnp.concatenate(ks + [km_ref[...]], axis=0)
        vals = jnp.concatenate(vs + [vm_ref[...]], axis=0)
        qtile = jnp.concatenate([q] * NH, axis=0)
        q_stack = jnp.where(lane == rowh, qtile, jnp.bfloat16(0.0))
        qk = jax.lax.dot_general(q_stack, keys, (((1,), (1,)), ((), ())),
                                 preferred_element_type=jnp.float32)
        e = jnp.exp(qk * scale)
        s = jnp.sum(e, axis=1, keepdims=True)
        pv = jax.lax.dot_general(e.astype(jnp.bfloat16), vals,
                                 (((1,), (0,)), ((), ())),
                                 preferred_element_type=jnp.float32)
        pv = pv * (1.0 / s)
        msg = jnp.zeros((WS, D), jnp.float32)
        for h in range(NH):
            msg = msg + jnp.where(lane49 == h,
                                  pv[h * WS:(h + 1) * WS, :], 0.0)
        msgs.append(msg)
    msg = jnp.concatenate(msgs, axis=0)
    xb = x_ref[...]
    msg = _dotTbf(msg, wm_ref[...])
    msg = _layer_norm(msg, g1_ref[...], b1_ref[...])
    hcat = jnp.concatenate([xb, msg], axis=1)
    h1 = jnp.maximum(_dotTbf(hcat, w1_ref[...]), 0.0)
    h2 = _dotTbf(h1, w2_ref[...])
    out_ref[...] = xb + _layer_norm(h2, g2_ref[...], b2_ref[...])


def kernel(x, source, Wq, Wk, Wv, Wm, W1, W2, g1, b1, g2, b2):
    b, d, H, Wd = x.shape
    m, n = H // WW, Wd // WW

    def to_win(t):
        t = t.reshape(d, m, WW, n, WW)
        t = jnp.transpose(t, (1, 3, 2, 4, 0))
        return t.reshape(NW * WS, d)

    xw = to_win(x[0])
    sw = to_win(source[0])
    bf = jnp.bfloat16

    blk = lambda i: (i, 0)
    qf, kf, vf, qm, km, vm, kmb, vmb = pl.pallas_call(
        _proj_kernel,
        grid=(N_BLKS,),
        in_specs=[
            pl.BlockSpec((ROWS_PER_BLK, D), blk),
            pl.BlockSpec((ROWS_PER_BLK, D), blk),
            pl.BlockSpec((D, D), lambda i: (0, 0)),
            pl.BlockSpec((D, D), lambda i: (0, 0)),
            pl.BlockSpec((D, D), lambda i: (0, 0)),
        ],
        out_specs=[
            pl.BlockSpec((ROWS_PER_BLK, D), blk),
            pl.BlockSpec((ROWS_PER_BLK, D), blk),
            pl.BlockSpec((ROWS_PER_BLK, D), blk),
            pl.BlockSpec((8, D), blk),
            pl.BlockSpec((8, D), blk),
            pl.BlockSpec((8, D), blk),
            pl.BlockSpec((8, D), blk),
            pl.BlockSpec((8, D), blk),
        ],
        out_shape=[
            jax.ShapeDtypeStruct((NW * WS, D), bf),
            jax.ShapeDtypeStruct((NW * WS, D), bf),
            jax.ShapeDtypeStruct((NW * WS, D), bf),
            jax.ShapeDtypeStruct((NW, D), jnp.float32),
            jax.ShapeDtypeStruct((NW, D), jnp.float32),
            jax.ShapeDtypeStruct((NW, D), jnp.float32),
            jax.ShapeDtypeStruct((NW, D), bf),
            jax.ShapeDtypeStruct((NW, D), bf),
        ],
    )(xw, sw, Wq.astype(bf), Wk.astype(bf), Wv.astype(bf))

    idx = pl.pallas_call(
        _topk_kernel,
        out_shape=jax.ShapeDtypeStruct((TK, NW), jnp.int32),
    )(km, qm)

    full2 = lambda i, s: (0, 0)
    rows = WPS * WS
    out_w = pl.pallas_call(
        _attn_kernel,
        grid_spec=pltpu.PrefetchScalarGridSpec(
            num_scalar_prefetch=1,
            grid=(NW // WPS,),
            in_specs=[
                pl.BlockSpec((rows, D), lambda i, s: (i, 0)),
                pl.BlockSpec((rows, D), lambda i, s: (i, 0)),
                pl.BlockSpec((NW, WS, D), lambda i, s: (0, 0, 0)),
                pl.BlockSpec((NW, WS, D), lambda i, s: (0, 0, 0)),
                pl.BlockSpec((NW, D), full2),
                pl.BlockSpec((NW, D), full2),
                pl.BlockSpec((D, D), full2),
                pl.BlockSpec((2 * D, 2 * D), full2),
                pl.BlockSpec((D, 2 * D), full2),
                pl.BlockSpec((1, D), full2),
                pl.BlockSpec((1, D), full2),
                pl.BlockSpec((1, D), full2),
                pl.BlockSpec((1, D), full2),
            ],
            out_specs=pl.BlockSpec((rows, D), lambda i, s: (i, 0)),
        ),
        out_shape=jax.ShapeDtypeStruct((NW * WS, D), jnp.float32),
    )(idx,
      qf, xw, kf.reshape(NW, WS, D), vf.reshape(NW, WS, D),
      kmb, vmb, Wm.astype(bf), W1.astype(bf), W2.astype(bf),
      g1.reshape(1, D), b1.reshape(1, D), g2.reshape(1, D), b2.reshape(1, D))

    out = out_w.reshape(m, n, WW, WW, d)
    out = jnp.transpose(out, (4, 0, 2, 1, 3)).reshape(1, d, H, Wd)
    return out
